# Initial kernel scaffold; baseline (speedup 1.0000x reference)
import jax
import jax.numpy as jnp
from jax.experimental import pallas as pl


def kernel(node_features, edge_index, edge_type, num_nodes, W_proj, b_proj, msg_W, msg_b, edge_tab, gru_Wih, gru_bih, gru_Whh, gru_bhh):
    raise NotImplementedError("write your pallas kernel here")


# speedup vs baseline: 111.7488x; 111.7488x over previous
import jax
import jax.numpy as jnp
import numpy as np
from jax import lax
from jax.experimental import pallas as pl
from jax.experimental.pallas import tpu as pltpu
from jax.experimental.pallas import tpu_sc as plsc

B, MAXN, MAXE = 8, 2048, 32768
DF, DH, L, NET = 128, 128, 3, 8

NC, NS = 2, 16
GPC = B // NC
CH = 128
EPT = MAXE // NS
NCH = EPT // CH
ROWS = MAXN + 32
ACC = GPC * ROWS
_Z = np.int32(0)


def _sc_body(table, srcg, dste, za, a_out,
             idx_src, idx_dst, rowbuf, a_acc, sem0, sem1):
    c = lax.axis_index("c")
    s = lax.axis_index("s")
    sems = (sem0, sem1)
    i32 = np.int32

    zshare = ACC // NS
    pltpu.sync_copy(za.at[pl.ds(s * zshare, zshare)],
                    a_acc.at[pl.ds(s * zshare, zshare)])
    plsc.subcore_barrier()

    for g in range(GPC):
        b = c * GPC + g
        pltpu.sync_copy(srcg.at[b, s], idx_src)
        pltpu.sync_copy(dste.at[b, s], idx_dst)
        descs = [None, None]
        descs[0] = pltpu.async_copy(table.at[idx_src.at[i32(0)]],
                                    rowbuf.at[i32(0)], sems[0])
        for j in range(NCH):
            if j + 1 < NCH:
                descs[(j + 1) % 2] = pltpu.async_copy(
                    table.at[idx_src.at[i32(j + 1)]],
                    rowbuf.at[i32((j + 1) % 2)],
                    sems[(j + 1) % 2])
            descs[j % 2].wait()
            pltpu.sync_copy(rowbuf.at[i32(j % 2)],
                            a_acc.at[idx_dst.at[i32(j)]], add=True)

    plsc.subcore_barrier()

    for g in range(GPC):
        b = c * GPC + g
        pltpu.sync_copy(a_acc.at[pl.ds(g * ROWS + s * 128, 128)],
                        a_out.at[pl.ds(b * MAXN + s * 128, 128)])


def _sc_scatter(table, srcg, dste, za):
    mesh = plsc.VectorSubcoreMesh(core_axis_name="c", subcore_axis_name="s",
                                  num_cores=NC, num_subcores=NS)
    return pl.kernel(
        _sc_body,
        out_type=jax.ShapeDtypeStruct((B * MAXN, DH), jnp.float32),
        mesh=mesh,
        scratch_types=[
            pltpu.VMEM((NCH, CH), jnp.int32),
            pltpu.VMEM((NCH, CH), jnp.int32),
            pltpu.VMEM((2, CH, DH), jnp.float32),
            pltpu.VMEM_SHARED((ACC, DH), jnp.float32),
            pltpu.SemaphoreType.DMA,
            pltpu.SemaphoreType.DMA,
        ],
        name="ggnn_sc_scatter",
    )(table, srcg, dste, za)


def _proj_body(x_ref, wt_ref, b_ref, o_ref):
    o_ref[...] = (jnp.dot(x_ref[...], wt_ref[...],
                          preferred_element_type=jnp.float32,
                          precision=lax.Precision.HIGHEST) + b_ref[...])


def _project(x_flat, wpt, bp):
    return pl.pallas_call(
        _proj_body,
        grid=(B,),
        in_specs=[
            pl.BlockSpec((MAXN, DF), lambda i: (i, _Z)),
            pl.BlockSpec((DF, DH), lambda i: (_Z, _Z)),
            pl.BlockSpec((1, DH), lambda i: (_Z, _Z)),
        ],
        out_specs=pl.BlockSpec((MAXN, DH), lambda i: (i, _Z)),
        out_shape=jax.ShapeDtypeStruct((B * MAXN, DH), jnp.float32),
    )(x_flat, wpt, bp)


def _layer_body(nn_ref, a_ref, c_ref, h_ref, mwt_ref, etab_ref, wiht_ref,
                whht_ref, bih_ref, bhh_ref, ho_ref, sum_ref):
    i = pl.program_id(0)
    h = h_ref[...]
    msgs = (jnp.dot(a_ref[...], mwt_ref[...],
                    preferred_element_type=jnp.float32,
                          precision=lax.Precision.HIGHEST)
            + jnp.dot(c_ref[...], etab_ref[...],
                      preferred_element_type=jnp.float32,
                          precision=lax.Precision.HIGHEST))
    gi = jnp.dot(msgs, wiht_ref[...],
                 preferred_element_type=jnp.float32,
                          precision=lax.Precision.HIGHEST) + bih_ref[...]
    gh = jnp.dot(h, whht_ref[...],
                 preferred_element_type=jnp.float32,
                          precision=lax.Precision.HIGHEST) + bhh_ref[...]
    r = jax.nn.sigmoid(gi[:, 0:DH] + gh[:, 0:DH])
    z = jax.nn.sigmoid(gi[:, DH:2 * DH] + gh[:, DH:2 * DH])
    ng = jnp.tanh(gi[:, 2 * DH:3 * DH] + r * gh[:, 2 * DH:3 * DH])
    hn = (1.0 - z) * ng + z * h
    has_edges = jnp.sum(c_ref[...]) > 0.5
    ho = jnp.where(has_edges, hn, h)
    ho_ref[...] = ho
    n = nn_ref[i]
    mask = lax.broadcasted_iota(jnp.int32, (MAXN, 1), 0) < n
    sum_ref[0, ...] = jnp.sum(jnp.where(mask, ho, 0.0), axis=0,
                              keepdims=True)


def _layer(nn, a_flat, c_flat, h_flat, mwt, etab, wiht, whht, bih, bhh):
    return pl.pallas_call(
        _layer_body,
        grid=(B,),
        in_specs=[
            pl.BlockSpec((B,), lambda i: (_Z,), memory_space=pltpu.SMEM),
            pl.BlockSpec((MAXN, DH), lambda i: (i, _Z)),
            pl.BlockSpec((MAXN, DH), lambda i: (i, _Z)),
            pl.BlockSpec((MAXN, DH), lambda i: (i, _Z)),
            pl.BlockSpec((DH, DH), lambda i: (_Z, _Z)),
            pl.BlockSpec((DH, DH), lambda i: (_Z, _Z)),
            pl.BlockSpec((DH, 3 * DH), lambda i: (_Z, _Z)),
            pl.BlockSpec((DH, 3 * DH), lambda i: (_Z, _Z)),
            pl.BlockSpec((1, 3 * DH), lambda i: (_Z, _Z)),
            pl.BlockSpec((1, 3 * DH), lambda i: (_Z, _Z)),
        ],
        out_specs=[
            pl.BlockSpec((MAXN, DH), lambda i: (i, _Z)),
            pl.BlockSpec((1, 1, DH), lambda i: (i, _Z, _Z)),
        ],
        out_shape=[
            jax.ShapeDtypeStruct((B * MAXN, DH), jnp.float32),
            jax.ShapeDtypeStruct((B, 1, DH), jnp.float32),
        ],
    )(nn, a_flat, c_flat, h_flat, mwt, etab, wiht, whht, bih, bhh)


def kernel(node_features, edge_index, edge_type, num_nodes, W_proj, b_proj,
           msg_W, msg_b, edge_tab, gru_Wih, gru_bih, gru_Whh, gru_bhh):
    f32 = jnp.float32
    i32 = jnp.int32
    nf = node_features.astype(f32).reshape(B * MAXN, DF)
    src = edge_index[:, 0, :].astype(i32)
    dst = edge_index[:, 1, :].astype(i32)
    et = jnp.clip(edge_type, 0, NET).astype(i32)
    n32 = num_nodes.astype(i32)

    valid = (src < n32[:, None]) & (dst < n32[:, None])
    boff = (jnp.arange(B, dtype=i32) * MAXN)[:, None]
    goff = ((jnp.arange(B, dtype=i32) % GPC) * ROWS)[:, None]
    srcg = (src + boff).reshape(B, NS, NCH, CH)
    dste = (jnp.where(valid, dst, MAXN) + goff).reshape(B, NS, NCH, CH)
    etx = et.reshape(B, NS, NCH, CH)
    onehot = jnp.eye(16, DH, dtype=f32)
    za = jnp.zeros((ACC, DH), f32)

    wpt = W_proj.astype(f32).T
    bp = b_proj.astype(f32).reshape(1, DH)
    mwt = msg_W.astype(f32)
    etab = jnp.zeros((L, DH, DH), f32).at[:, :NET + 1, :].set(
        edge_tab.astype(f32) + msg_b.astype(f32)[:, None, :])
    wiht = gru_Wih.astype(f32)
    whht = gru_Whh.astype(f32)
    bih = gru_bih.astype(f32).reshape(L, 1, 3 * DH)
    bhh = gru_bhh.astype(f32).reshape(L, 1, 3 * DH)

    h = _project(nf, wpt, bp)
    c_flat = _sc_scatter(onehot, etx, dste, za)
    out = None
    for l in range(L):
        a_flat = _sc_scatter(h, srcg, dste, za)
        h, out = _layer(n32, a_flat, c_flat, h, mwt[l].T, etab[l],
                        wiht[l].T, whht[l].T, bih[l], bhh[l])
    return out.reshape(B, DH).astype(jnp.float64)


# speedup vs baseline: 235.7582x; 2.1097x over previous
import jax
import jax.numpy as jnp
import numpy as np
from jax import lax
from jax.experimental import pallas as pl
from jax.experimental.pallas import tpu as pltpu
from jax.experimental.pallas import tpu_sc as plsc

B, MAXN, MAXE = 8, 2048, 32768
DF, DH, L, NET = 128, 128, 3, 8

NC, NS = 2, 16
GPC = B // NC
CH = 128
EPT = MAXE // NS
NCH = EPT // CH
ROWS = MAXN + 32
ACC = GPC * ROWS
_Z = np.int32(0)


def _sc_body(table, srcg, dste, za, a_out,
             idx_src, idx_dst, rowbuf, a_acc, sem0, sem1):
    c = lax.axis_index("c")
    s = lax.axis_index("s")
    sems = (sem0, sem1)
    i32 = np.int32

    zshare = ACC // NS
    pltpu.sync_copy(za.at[pl.ds(s * zshare, zshare)],
                    a_acc.at[pl.ds(s * zshare, zshare)])
    plsc.subcore_barrier()

    for g in range(GPC):
        b = c * GPC + g
        pltpu.sync_copy(srcg.at[b, s], idx_src)
        pltpu.sync_copy(dste.at[b, s], idx_dst)
        descs = [None, None]
        descs[0] = pltpu.async_copy(table.at[idx_src.at[i32(0)]],
                                    rowbuf.at[i32(0)], sems[0])
        for j in range(NCH):
            if j + 1 < NCH:
                descs[(j + 1) % 2] = pltpu.async_copy(
                    table.at[idx_src.at[i32(j + 1)]],
                    rowbuf.at[i32((j + 1) % 2)],
                    sems[(j + 1) % 2])
            descs[j % 2].wait()
            pltpu.sync_copy(rowbuf.at[i32(j % 2)],
                            a_acc.at[idx_dst.at[i32(j)]], add=True)

    plsc.subcore_barrier()

    for g in range(GPC):
        b = c * GPC + g
        pltpu.sync_copy(a_acc.at[pl.ds(g * ROWS + s * 128, 128)],
                        a_out.at[pl.ds(b * MAXN + s * 128, 128)])


def _sc_scatter(table, srcg, dste, za):
    mesh = plsc.VectorSubcoreMesh(core_axis_name="c", subcore_axis_name="s",
                                  num_cores=NC, num_subcores=NS)
    return pl.kernel(
        _sc_body,
        out_type=jax.ShapeDtypeStruct((B * MAXN, DH), jnp.float32),
        mesh=mesh,
        scratch_types=[
            pltpu.VMEM((NCH, CH), jnp.int32),
            pltpu.VMEM((NCH, CH), jnp.int32),
            pltpu.VMEM((2, CH, DH), jnp.float32),
            pltpu.VMEM_SHARED((ACC, DH), jnp.float32),
            pltpu.SemaphoreType.DMA,
            pltpu.SemaphoreType.DMA,
        ],
        name="ggnn_sc_scatter",
    )(table, srcg, dste, za)


def _proj_body(x_ref, wt_ref, b_ref, o_ref):
    o_ref[...] = (jnp.dot(x_ref[...], wt_ref[...],
                          preferred_element_type=jnp.float32,
                          precision=lax.Precision.HIGHEST) + b_ref[...])


def _project(x_flat, wpt, bp):
    return pl.pallas_call(
        _proj_body,
        grid=(B,),
        in_specs=[
            pl.BlockSpec((MAXN, DF), lambda i: (i, _Z)),
            pl.BlockSpec((DF, DH), lambda i: (_Z, _Z)),
            pl.BlockSpec((1, DH), lambda i: (_Z, _Z)),
        ],
        out_specs=pl.BlockSpec((MAXN, DH), lambda i: (i, _Z)),
        out_shape=jax.ShapeDtypeStruct((B * MAXN, DH), jnp.float32),
    )(x_flat, wpt, bp)


def _layer_body(nn_ref, a_ref, c_ref, h_ref, mwt_ref, etab_ref, wiht_ref,
                whht_ref, bih_ref, bhh_ref, ho_ref, sum_ref):
    i = pl.program_id(0)
    h = h_ref[...]
    msgs = (jnp.dot(a_ref[...], mwt_ref[...],
                    preferred_element_type=jnp.float32,
                          precision=lax.Precision.HIGHEST)
            + jnp.dot(c_ref[...], etab_ref[...],
                      preferred_element_type=jnp.float32,
                          precision=lax.Precision.HIGHEST))
    gi = jnp.dot(msgs, wiht_ref[...],
                 preferred_element_type=jnp.float32,
                          precision=lax.Precision.HIGHEST) + bih_ref[...]
    gh = jnp.dot(h, whht_ref[...],
                 preferred_element_type=jnp.float32,
                          precision=lax.Precision.HIGHEST) + bhh_ref[...]
    r = jax.nn.sigmoid(gi[:, 0:DH] + gh[:, 0:DH])
    z = jax.nn.sigmoid(gi[:, DH:2 * DH] + gh[:, DH:2 * DH])
    ng = jnp.tanh(gi[:, 2 * DH:3 * DH] + r * gh[:, 2 * DH:3 * DH])
    hn = (1.0 - z) * ng + z * h
    has_edges = jnp.sum(c_ref[...]) > 0.5
    ho = jnp.where(has_edges, hn, h)
    ho_ref[...] = ho
    n = nn_ref[i]
    mask = lax.broadcasted_iota(jnp.int32, (MAXN, 1), 0) < n
    sum_ref[0, ...] = jnp.sum(jnp.where(mask, ho, 0.0), axis=0,
                              keepdims=True)


def _layer(nn, a_flat, c_flat, h_flat, mwt, etab, wiht, whht, bih, bhh):
    return pl.pallas_call(
        _layer_body,
        grid=(B,),
        in_specs=[
            pl.BlockSpec((B,), lambda i: (_Z,), memory_space=pltpu.SMEM),
            pl.BlockSpec((MAXN, DH), lambda i: (i, _Z)),
            pl.BlockSpec((MAXN, DH), lambda i: (i, _Z)),
            pl.BlockSpec((MAXN, DH), lambda i: (i, _Z)),
            pl.BlockSpec((DH, DH), lambda i: (_Z, _Z)),
            pl.BlockSpec((DH, DH), lambda i: (_Z, _Z)),
            pl.BlockSpec((DH, 3 * DH), lambda i: (_Z, _Z)),
            pl.BlockSpec((DH, 3 * DH), lambda i: (_Z, _Z)),
            pl.BlockSpec((1, 3 * DH), lambda i: (_Z, _Z)),
            pl.BlockSpec((1, 3 * DH), lambda i: (_Z, _Z)),
        ],
        out_specs=[
            pl.BlockSpec((MAXN, DH), lambda i: (i, _Z)),
            pl.BlockSpec((1, 1, DH), lambda i: (i, _Z, _Z)),
        ],
        out_shape=[
            jax.ShapeDtypeStruct((B * MAXN, DH), jnp.float32),
            jax.ShapeDtypeStruct((B, 1, DH), jnp.float32),
        ],
    )(nn, a_flat, c_flat, h_flat, mwt, etab, wiht, whht, bih, bhh)


def kernel(node_features, edge_index, edge_type, num_nodes, W_proj, b_proj,
           msg_W, msg_b, edge_tab, gru_Wih, gru_bih, gru_Whh, gru_bhh):
    f32 = jnp.float32
    i32 = jnp.int32
    nf = node_features.astype(f32).reshape(B * MAXN, DF)
    src = edge_index[:, 0, :].astype(i32)
    dst = edge_index[:, 1, :].astype(i32)
    et = jnp.clip(edge_type, 0, NET).astype(i32)
    n32 = num_nodes.astype(i32)

    valid = (src < n32[:, None]) & (dst < n32[:, None])
    boff = (jnp.arange(B, dtype=i32) * MAXN)[:, None]
    goff = ((jnp.arange(B, dtype=i32) % GPC) * ROWS)[:, None]
    srcg = (src + boff).reshape(B, NS, NCH, CH)
    dste = (jnp.where(valid, dst, MAXN) + goff).reshape(B, NS, NCH, CH)
    rep = 256
    spread = (jnp.arange(MAXE, dtype=i32) % rep) * 16
    etx = (et + spread[None, :]).reshape(B, NS, NCH, CH)
    onehot = jnp.tile(jnp.eye(16, DH, dtype=f32), (rep, 1))
    za = jnp.zeros((ACC, DH), f32)

    wpt = W_proj.astype(f32).T
    bp = b_proj.astype(f32).reshape(1, DH)
    mwt = msg_W.astype(f32)
    etab = jnp.zeros((L, DH, DH), f32).at[:, :NET + 1, :].set(
        edge_tab.astype(f32) + msg_b.astype(f32)[:, None, :])
    wiht = gru_Wih.astype(f32)
    whht = gru_Whh.astype(f32)
    bih = gru_bih.astype(f32).reshape(L, 1, 3 * DH)
    bhh = gru_bhh.astype(f32).reshape(L, 1, 3 * DH)

    h = _project(nf, wpt, bp)
    c_flat = _sc_scatter(onehot, etx, dste, za)
    out = None
    for l in range(L):
        a_flat = _sc_scatter(h, srcg, dste, za)
        h, out = _layer(n32, a_flat, c_flat, h, mwt[l].T, etab[l],
                        wiht[l].T, whht[l].T, bih[l], bhh[l])
    return out.reshape(B, DH).astype(jnp.float64)


# speedup vs baseline: 255.4876x; 1.0837x over previous
import jax
import jax.numpy as jnp
import numpy as np
from jax import lax
from jax.experimental import pallas as pl
from jax.experimental.pallas import tpu as pltpu
from jax.experimental.pallas import tpu_sc as plsc

B, MAXN, MAXE = 8, 2048, 32768
DF, DH, L, NET = 128, 128, 3, 8

NC, NS = 2, 16
GPC = B // NC
CH = 128
EPT = MAXE // NS
NCH = EPT // CH
ROWS = MAXN + 32
ACC = GPC * ROWS
NBUF = 3
_Z = np.int32(0)


def _sc_body(table, srcg, dste, za, a_out,
             idx_src, idx_dst, rowbuf, a_acc, *allsems):
    c = lax.axis_index("c")
    s = lax.axis_index("s")
    sems = allsems[:NBUF]
    ssems = allsems[NBUF:]
    i32 = np.int32

    zshare = ACC // NS
    pltpu.sync_copy(za.at[pl.ds(s * zshare, zshare)],
                    a_acc.at[pl.ds(s * zshare, zshare)])
    plsc.subcore_barrier()

    for g in range(GPC):
        b = c * GPC + g
        pltpu.sync_copy(srcg.at[b, s], idx_src)
        pltpu.sync_copy(dste.at[b, s], idx_dst)
        gdesc = [None] * NBUF
        sdesc = [None] * NBUF
        for j in range(NBUF - 1):
            gdesc[j] = pltpu.async_copy(table.at[idx_src.at[i32(j)]],
                                        rowbuf.at[i32(j)], sems[j])
        for j in range(NCH):
            k = j % NBUF
            gdesc[k].wait()
            sdesc[k] = pltpu.async_copy(rowbuf.at[i32(k)],
                                        a_acc.at[idx_dst.at[i32(j)]],
                                        ssems[k], add=True)
            jn = j + NBUF - 1
            if jn < NCH:
                kp = jn % NBUF
                if sdesc[kp] is not None:
                    sdesc[kp].wait()
                    sdesc[kp] = None
                gdesc[kp] = pltpu.async_copy(
                    table.at[idx_src.at[i32(jn)]],
                    rowbuf.at[i32(kp)], sems[kp])
        for k in range(NBUF):
            if sdesc[k] is not None:
                sdesc[k].wait()

    plsc.subcore_barrier()

    for g in range(GPC):
        b = c * GPC + g
        pltpu.sync_copy(a_acc.at[pl.ds(g * ROWS + s * 128, 128)],
                        a_out.at[pl.ds(b * MAXN + s * 128, 128)])


def _sc_scatter(table, srcg, dste, za):
    mesh = plsc.VectorSubcoreMesh(core_axis_name="c", subcore_axis_name="s",
                                  num_cores=NC, num_subcores=NS)
    return pl.kernel(
        _sc_body,
        out_type=jax.ShapeDtypeStruct((B * MAXN, DH), jnp.float32),
        mesh=mesh,
        scratch_types=[
            pltpu.VMEM((NCH, CH), jnp.int32),
            pltpu.VMEM((NCH, CH), jnp.int32),
            pltpu.VMEM((NBUF, CH, DH), jnp.float32),
            pltpu.VMEM_SHARED((ACC, DH), jnp.float32),
        ] + [pltpu.SemaphoreType.DMA] * (2 * NBUF),
        name="ggnn_sc_scatter",
    )(table, srcg, dste, za)


def _proj_body(x_ref, wt_ref, b_ref, o_ref):
    o_ref[...] = (jnp.dot(x_ref[...], wt_ref[...],
                          preferred_element_type=jnp.float32,
                          precision=lax.Precision.HIGHEST) + b_ref[...])


def _project(x_flat, wpt, bp):
    return pl.pallas_call(
        _proj_body,
        grid=(B,),
        in_specs=[
            pl.BlockSpec((MAXN, DF), lambda i: (i, _Z)),
            pl.BlockSpec((DF, DH), lambda i: (_Z, _Z)),
            pl.BlockSpec((1, DH), lambda i: (_Z, _Z)),
        ],
        out_specs=pl.BlockSpec((MAXN, DH), lambda i: (i, _Z)),
        out_shape=jax.ShapeDtypeStruct((B * MAXN, DH), jnp.float32),
    )(x_flat, wpt, bp)


def _layer_body(nn_ref, a_ref, c_ref, h_ref, mwt_ref, etab_ref, wiht_ref,
                whht_ref, bsum_ref, bhhn_ref, ho_ref, sum_ref):
    i = pl.program_id(0)
    h = h_ref[...]
    msgs = jnp.dot(jnp.concatenate([a_ref[...], c_ref[...]], axis=1),
                   jnp.concatenate([mwt_ref[...], etab_ref[...]], axis=0),
                   preferred_element_type=jnp.float32,
                   precision=lax.Precision.HIGHEST)
    girh = jnp.dot(jnp.concatenate([msgs, h], axis=1),
                   jnp.concatenate([wiht_ref[...], whht_ref[...]], axis=0),
                   preferred_element_type=jnp.float32,
                   precision=lax.Precision.HIGHEST) + bsum_ref[...]
    ghn = jnp.dot(h, whht_ref[:, 2 * DH:3 * DH],
                  preferred_element_type=jnp.float32,
                  precision=lax.Precision.HIGHEST) + bhhn_ref[...]
    r = jax.nn.sigmoid(girh[:, 0:DH])
    z = jax.nn.sigmoid(girh[:, DH:2 * DH])
    ng = jnp.tanh(girh[:, 2 * DH:3 * DH] + (r - 1.0) * ghn)
    hn = (1.0 - z) * ng + z * h
    has_edges = jnp.sum(c_ref[...]) > 0.5
    ho = jnp.where(has_edges, hn, h)
    ho_ref[...] = ho
    n = nn_ref[i]
    mask = lax.broadcasted_iota(jnp.int32, (MAXN, 1), 0) < n
    sum_ref[0, ...] = jnp.sum(jnp.where(mask, ho, 0.0), axis=0,
                              keepdims=True)


def _layer(nn, a_flat, c_flat, h_flat, mwt, etab, wiht, whht, bsum, bhhn):
    return pl.pallas_call(
        _layer_body,
        grid=(B,),
        in_specs=[
            pl.BlockSpec((B,), lambda i: (_Z,), memory_space=pltpu.SMEM),
            pl.BlockSpec((MAXN, DH), lambda i: (i, _Z)),
            pl.BlockSpec((MAXN, DH), lambda i: (i, _Z)),
            pl.BlockSpec((MAXN, DH), lambda i: (i, _Z)),
            pl.BlockSpec((DH, DH), lambda i: (_Z, _Z)),
            pl.BlockSpec((DH, DH), lambda i: (_Z, _Z)),
            pl.BlockSpec((DH, 3 * DH), lambda i: (_Z, _Z)),
            pl.BlockSpec((DH, 3 * DH), lambda i: (_Z, _Z)),
            pl.BlockSpec((1, 3 * DH), lambda i: (_Z, _Z)),
            pl.BlockSpec((1, DH), lambda i: (_Z, _Z)),
        ],
        out_specs=[
            pl.BlockSpec((MAXN, DH), lambda i: (i, _Z)),
            pl.BlockSpec((1, 1, DH), lambda i: (i, _Z, _Z)),
        ],
        out_shape=[
            jax.ShapeDtypeStruct((B * MAXN, DH), jnp.float32),
            jax.ShapeDtypeStruct((B, 1, DH), jnp.float32),
        ],
    )(nn, a_flat, c_flat, h_flat, mwt, etab, wiht, whht, bsum, bhhn)


def kernel(node_features, edge_index, edge_type, num_nodes, W_proj, b_proj,
           msg_W, msg_b, edge_tab, gru_Wih, gru_bih, gru_Whh, gru_bhh):
    f32 = jnp.float32
    i32 = jnp.int32
    nf = node_features.astype(f32).reshape(B * MAXN, DF)
    src = edge_index[:, 0, :].astype(i32)
    dst = edge_index[:, 1, :].astype(i32)
    et = jnp.clip(edge_type, 0, NET).astype(i32)
    n32 = num_nodes.astype(i32)

    valid = (src < n32[:, None]) & (dst < n32[:, None])
    boff = (jnp.arange(B, dtype=i32) * MAXN)[:, None]
    goff = ((jnp.arange(B, dtype=i32) % GPC) * ROWS)[:, None]
    srcg = (src + boff).reshape(B, NS, NCH, CH)
    dste = (jnp.where(valid, dst, MAXN) + goff).reshape(B, NS, NCH, CH)
    rep = 256
    spread = (jnp.arange(MAXE, dtype=i32) % rep) * 16
    etx = (et + spread[None, :]).reshape(B, NS, NCH, CH)
    onehot = jnp.tile(jnp.eye(16, DH, dtype=f32), (rep, 1))
    za = jnp.zeros((ACC, DH), f32)

    wpt = W_proj.astype(f32).T
    bp = b_proj.astype(f32).reshape(1, DH)
    mwt = msg_W.astype(f32)
    etab = jnp.zeros((L, DH, DH), f32).at[:, :NET + 1, :].set(
        edge_tab.astype(f32) + msg_b.astype(f32)[:, None, :])
    wiht = gru_Wih.astype(f32)
    whht = gru_Whh.astype(f32)
    bsum = (gru_bih.astype(f32) + gru_bhh.astype(f32)).reshape(L, 1, 3 * DH)
    bhhn = gru_bhh.astype(f32)[:, 2 * DH:3 * DH].reshape(L, 1, DH)

    h = _project(nf, wpt, bp)
    c_flat = _sc_scatter(onehot, etx, dste, za)
    out = None
    for l in range(L):
        a_flat = _sc_scatter(h, srcg, dste, za)
        h, out = _layer(n32, a_flat, c_flat, h, mwt[l].T, etab[l],
                        wiht[l].T, whht[l].T, bsum[l], bhhn[l])
    return out.reshape(B, DH).astype(jnp.float64)
